# baseline (speedup 1.0000x reference)
import functools

import jax
import jax.numpy as jnp
from jax import lax
from jax.experimental import pallas as pl
from jax.experimental.pallas import tpu as pltpu
from jax.experimental.pallas import tpu_sc as plsc

B = 16384
D = 2048
NROWS = 12
L = 16
NC = 2
NS = 16
NW = NC * NS
B_PER_W = B // NW
C = 8
NCHUNK = B_PER_W // C
NBUF = 4
LOOKAHEAD = 3


def _sc_add_lookup(noise, ids, table):
    mesh = plsc.VectorSubcoreMesh(core_axis_name="c", subcore_axis_name="s")

    @functools.partial(
        pl.kernel,
        out_type=jax.ShapeDtypeStruct((B, D), jnp.float32),
        mesh=mesh,
        compiler_params=pltpu.CompilerParams(needs_layout_passes=False),
        scratch_types=[
            pltpu.VMEM((B_PER_W,), jnp.int32),
            pltpu.VMEM((NROWS, D), jnp.float32),
            [pltpu.VMEM((C, D), jnp.float32) for _ in range(NBUF)],
            [pltpu.SemaphoreType.DMA for _ in range(NBUF)],
            [pltpu.SemaphoreType.DMA for _ in range(NBUF)],
        ],
    )
    def k(noise_hbm, ids_hbm, table_hbm, out_hbm, idx_v, table_v, bufs,
          sems_in, sems_out):
        wid = lax.axis_index("s") * NC + lax.axis_index("c")
        base = wid * B_PER_W
        iota = lax.iota(jnp.int32, L)

        def start_in(g, b):
            pltpu.async_copy(noise_hbm.at[pl.ds(base + g * C, C)], bufs[b],
                             sems_in[b])

        def wait_in(b):
            pltpu.make_async_copy(noise_hbm.at[pl.ds(base, C)], bufs[b],
                                  sems_in[b]).wait()

        def start_out(g, b):
            pltpu.async_copy(bufs[b], out_hbm.at[pl.ds(base + g * C, C)],
                             sems_out[b])

        def wait_out(b):
            pltpu.make_async_copy(bufs[b], out_hbm.at[pl.ds(base, C)],
                                  sems_out[b]).wait()

        for b in range(LOOKAHEAD):
            start_in(b, b)
        pltpu.sync_copy(ids_hbm.at[pl.ds(base, B_PER_W)], idx_v)
        pltpu.sync_copy(table_hbm, table_v)

        def accumulate(buf, g):
            idvec = idx_v[pl.ds(lax.div(g, L // C) * L, L)]
            lane0 = lax.rem(g, L // C) * C
            rids = [jnp.max(jnp.where(iota == lane0 + r, idvec, 0))
                    for r in range(C)]

            @plsc.parallel_loop(0, D // L, 1, unroll=4)
            def _(j):
                for r in range(C):
                    tbl = table_v[rids[r], pl.ds(j * L, L)]
                    plsc.addupdate(buf.at[r, pl.ds(j * L, L)], tbl)

        def round_body(p, carry):
            for b in range(NBUF):
                g = p * NBUF + b
                wait_in(b)
                accumulate(bufs[b], g)
                start_out(g, b)
                bnext = (b + LOOKAHEAD) % NBUF

                @pl.when(g + LOOKAHEAD < NCHUNK)
                def _(g=g, bnext=bnext):
                    @pl.when(g >= NBUF - LOOKAHEAD)
                    def _():
                        wait_out(bnext)

                    start_in(g + LOOKAHEAD, bnext)

            return carry

        lax.fori_loop(0, NCHUNK // NBUF, round_body, 0)

        for g in range(NCHUNK - NBUF, NCHUNK):
            wait_out(g % NBUF)

    return k(noise, ids, table)


def kernel(noise_embedding, principle_ids, table):
    ids32 = principle_ids.astype(jnp.int32)
    out = _sc_add_lookup(noise_embedding, ids32, table)
    return (out, principle_ids)


# speedup vs baseline: 1.0071x; 1.0071x over previous
import functools

import jax
import jax.numpy as jnp
from jax import lax
from jax.experimental import pallas as pl
from jax.experimental.pallas import tpu as pltpu
from jax.experimental.pallas import tpu_sc as plsc

B = 16384
D = 2048
NROWS = 12
L = 16
NC = 2
NS = 16
NW = NC * NS
B_PER_W = B // NW
C = 8
NCHUNK = B_PER_W // C
NBUF = 4
LOOKAHEAD = 3


def _sc_add_lookup(noise, ids, table):
    mesh = plsc.VectorSubcoreMesh(core_axis_name="c", subcore_axis_name="s")

    @functools.partial(
        pl.kernel,
        out_type=jax.ShapeDtypeStruct((B, D), jnp.float32),
        mesh=mesh,
        compiler_params=pltpu.CompilerParams(needs_layout_passes=False),
        scratch_types=[
            pltpu.VMEM((B_PER_W,), jnp.int32),
            pltpu.VMEM((NROWS, D), jnp.float32),
            [pltpu.VMEM((C, D), jnp.float32) for _ in range(NBUF)],
            [pltpu.SemaphoreType.DMA for _ in range(NBUF)],
            [pltpu.SemaphoreType.DMA for _ in range(NBUF)],
        ],
    )
    def k(noise_hbm, ids_hbm, table_hbm, out_hbm, idx_v, table_v, bufs,
          sems_in, sems_out):
        wid = lax.axis_index("s") * NC + lax.axis_index("c")
        base = wid * B_PER_W
        iota = lax.iota(jnp.int32, L)

        def start_in(g, b):
            pltpu.async_copy(noise_hbm.at[pl.ds(base + g * C, C)], bufs[b],
                             sems_in[b])

        def wait_in(b):
            pltpu.make_async_copy(noise_hbm.at[pl.ds(base, C)], bufs[b],
                                  sems_in[b]).wait()

        def start_out(g, b):
            pltpu.async_copy(bufs[b], out_hbm.at[pl.ds(base + g * C, C)],
                             sems_out[b])

        def wait_out(b):
            pltpu.make_async_copy(bufs[b], out_hbm.at[pl.ds(base, C)],
                                  sems_out[b]).wait()

        for b in range(LOOKAHEAD):
            start_in(b, b)
        pltpu.sync_copy(ids_hbm.at[pl.ds(base, B_PER_W)], idx_v)
        pltpu.sync_copy(table_hbm, table_v)

        def accumulate(buf, g):
            idvec = idx_v[pl.ds(lax.div(g, L // C) * L, L)]
            lane0 = lax.rem(g, L // C) * C
            rids = [jnp.max(jnp.where(iota == lane0 + r, idvec, 0))
                    for r in range(C)]

            @plsc.parallel_loop(0, D // L, 1, unroll=2)
            def _(j):
                for r in range(C):
                    tbl = table_v[rids[r], pl.ds(j * L, L)]
                    plsc.addupdate(buf.at[r, pl.ds(j * L, L)], tbl)

        def round_body(p, carry):
            for b in range(NBUF):
                g = p * NBUF + b
                wait_in(b)
                accumulate(bufs[b], g)
                start_out(g, b)
                bnext = (b + LOOKAHEAD) % NBUF

                @pl.when(g + LOOKAHEAD < NCHUNK)
                def _(g=g, bnext=bnext):
                    @pl.when(g >= NBUF - LOOKAHEAD)
                    def _():
                        wait_out(bnext)

                    start_in(g + LOOKAHEAD, bnext)

            return carry

        lax.fori_loop(0, NCHUNK // NBUF, round_body, 0)

        for g in range(NCHUNK - NBUF, NCHUNK):
            wait_out(g % NBUF)

    return k(noise, ids, table)


def kernel(noise_embedding, principle_ids, table):
    ids32 = principle_ids.astype(jnp.int32)
    out = _sc_add_lookup(noise_embedding, ids32, table)
    return (out, principle_ids)
